# Initial kernel scaffold; baseline (speedup 1.0000x reference)
import jax
import jax.numpy as jnp
from jax.experimental import pallas as pl


def kernel(sample, item_id_table, category_table, brand_table, author_table, map_category, map_brand, map_author, W, b):
    raise NotImplementedError("write your pallas kernel here")


# speedup vs baseline: 8.2324x; 8.2324x over previous
import functools

import jax
import jax.numpy as jnp
from jax import lax
from jax.experimental import pallas as pl
from jax.experimental.pallas import tpu as pltpu
from jax.experimental.pallas import tpu_sc as plsc

NUM_IDS = 100000
PAD_IDS = 102400
N_FLAT = 4096 * 50
FINAL = 256
NC, NS = 2, 16
NW = NC * NS
CHUNK = 128

_MESH = dict(core_axis_name="c", subcore_axis_name="s")


def _wid():
    return lax.axis_index("s") * NC + lax.axis_index("c")


@functools.cache
def _make_attr_gather():
    return functools.partial(
        pl.kernel,
        mesh=plsc.VectorSubcoreMesh(**_MESH),
        out_type=[jax.ShapeDtypeStruct((PAD_IDS, 128), jnp.float32)] * 3,
        scratch_types=[
            pltpu.VMEM((CHUNK,), jnp.int32),
            pltpu.VMEM((CHUNK,), jnp.int32),
            pltpu.VMEM((CHUNK,), jnp.int32),
            pltpu.VMEM((CHUNK, 128), jnp.float32),
            pltpu.VMEM((CHUNK, 128), jnp.float32),
            pltpu.VMEM((CHUNK, 128), jnp.float32),
            pltpu.SemaphoreType.DMA,
        ],
    )(_attr_gather_body)


def _attr_gather_body(mc, mb, ma, ct, bt, at, xc, xb, xa,
                      idxc, idxb, idxa, rc, rb, ra, sem):
    cpw = PAD_IDS // (NW * CHUNK)
    wid = _wid()

    def body(i, carry):
        base = (wid * cpw + i) * CHUNK
        pltpu.sync_copy(mc.at[pl.ds(base, CHUNK)], idxc)
        pltpu.sync_copy(mb.at[pl.ds(base, CHUNK)], idxb)
        pltpu.sync_copy(ma.at[pl.ds(base, CHUNK)], idxa)
        cc = pltpu.async_copy(ct.at[idxc], rc, sem)
        cb = pltpu.async_copy(bt.at[idxb], rb, sem)
        ca = pltpu.async_copy(at.at[idxa], ra, sem)
        cc.wait()
        cb.wait()
        ca.wait()
        pltpu.sync_copy(rc, xc.at[pl.ds(base, CHUNK)])
        pltpu.sync_copy(rb, xb.at[pl.ds(base, CHUNK)])
        pltpu.sync_copy(ra, xa.at[pl.ds(base, CHUNK)])
        return carry

    lax.fori_loop(0, cpw, body, 0)


def _mm_body(item_ref, xc_ref, xb_ref, xa_ref, w_ref, b_ref, h_ref):
    acc = jnp.dot(item_ref[...], w_ref[0:128, :],
                  preferred_element_type=jnp.float32)
    acc = acc + jnp.dot(xc_ref[:, 0:64], w_ref[128:192, :],
                        preferred_element_type=jnp.float32)
    acc = acc + jnp.dot(xb_ref[:, 0:64], w_ref[192:256, :],
                        preferred_element_type=jnp.float32)
    acc = acc + jnp.dot(xa_ref[:, 0:64], w_ref[256:320, :],
                        preferred_element_type=jnp.float32)
    h = jnp.tanh(acc + b_ref[...])
    row = lax.broadcasted_iota(jnp.int32, h.shape, 0)
    h_ref[...] = jnp.where((pl.program_id(0) == 0) & (row == 0), 0.0, h)


def _make_mm(bm):
    grid = NUM_IDS // bm
    return pl.pallas_call(
        _mm_body,
        grid=(grid,),
        in_specs=[
            pl.BlockSpec((bm, 128), lambda i: (i, 0)),
            pl.BlockSpec((bm, 128), lambda i: (i, 0)),
            pl.BlockSpec((bm, 128), lambda i: (i, 0)),
            pl.BlockSpec((bm, 128), lambda i: (i, 0)),
            pl.BlockSpec((320, FINAL), lambda i: (0, 0)),
            pl.BlockSpec((1, FINAL), lambda i: (0, 0)),
        ],
        out_specs=pl.BlockSpec((bm, FINAL), lambda i: (i, 0)),
        out_shape=jax.ShapeDtypeStruct((NUM_IDS, FINAL), jnp.float32),
    )


@functools.cache
def _make_emb_gather():
    return functools.partial(
        pl.kernel,
        mesh=plsc.VectorSubcoreMesh(**_MESH),
        out_type=jax.ShapeDtypeStruct((N_FLAT, FINAL), jnp.float32),
        scratch_types=[
            pltpu.VMEM((CHUNK,), jnp.int32),
            pltpu.VMEM((CHUNK, FINAL), jnp.float32),
            pltpu.SemaphoreType.DMA,
        ],
    )(_emb_gather_body)


def _emb_gather_body(h, flat, out, idx, rows, sem):
    cpw = N_FLAT // (NW * CHUNK)
    wid = _wid()

    def body(i, carry):
        base = (wid * cpw + i) * CHUNK
        pltpu.sync_copy(flat.at[pl.ds(base, CHUNK)], idx)
        pltpu.async_copy(h.at[idx], rows, sem).wait()
        pltpu.sync_copy(rows, out.at[pl.ds(base, CHUNK)])
        return carry

    lax.fori_loop(0, cpw, body, 0)


def kernel(sample, item_id_table, category_table, brand_table, author_table,
           map_category, map_brand, map_author, W, b):
    pad = PAD_IDS - NUM_IDS
    mc = jnp.pad(map_category, (0, pad))
    mb = jnp.pad(map_brand, (0, pad))
    ma = jnp.pad(map_author, (0, pad))
    ct = jnp.pad(category_table, ((0, 0), (0, 64)))
    bt = jnp.pad(brand_table, ((0, 0), (0, 64)))
    at = jnp.pad(author_table, ((0, 0), (0, 64)))
    xc, xb, xa = _make_attr_gather()(mc, mb, ma, ct, bt, at)
    h = _make_mm(800)(item_id_table, xc, xb, xa, W, b.reshape(1, FINAL))
    flat = sample.reshape(-1).astype(jnp.int32)
    out = _make_emb_gather()(h, flat)
    return out.reshape(sample.shape + (FINAL,))


# speedup vs baseline: 9.9077x; 1.2035x over previous
import functools

import jax
import jax.numpy as jnp
from jax import lax
from jax.experimental import pallas as pl
from jax.experimental.pallas import tpu as pltpu
from jax.experimental.pallas import tpu_sc as plsc

NUM_IDS = 100000
PAD_IDS = 102400
N_FLAT = 4096 * 50
FINAL = 256
NC, NS = 2, 16
NW = NC * NS
CHUNK = 128

_MESH = dict(core_axis_name="c", subcore_axis_name="s")


def _wid():
    return lax.axis_index("s") * NC + lax.axis_index("c")


@functools.cache
def _make_attr_gather():
    return functools.partial(
        pl.kernel,
        mesh=plsc.VectorSubcoreMesh(**_MESH),
        out_type=[jax.ShapeDtypeStruct((PAD_IDS, 128), jnp.float32)] * 3,
        scratch_types=[
            pltpu.VMEM((PAD_IDS // NW,), jnp.int32),
            pltpu.VMEM((PAD_IDS // NW,), jnp.int32),
            pltpu.VMEM((PAD_IDS // NW,), jnp.int32),
            pltpu.VMEM((CHUNK, 128), jnp.float32),
            pltpu.VMEM((CHUNK, 128), jnp.float32),
            pltpu.VMEM((CHUNK, 128), jnp.float32),
            pltpu.VMEM((CHUNK, 128), jnp.float32),
            pltpu.VMEM((CHUNK, 128), jnp.float32),
            pltpu.VMEM((CHUNK, 128), jnp.float32),
            pltpu.SemaphoreType.DMA,
            pltpu.SemaphoreType.DMA,
        ],
    )(_attr_gather_body)


def _attr_gather_body(mc, mb, ma, ct, bt, at, xc, xb, xa,
                      ic, ib, ia, rc0, rb0, ra0, rc1, rb1, ra1, s0, s1):
    rpw = PAD_IDS // NW
    cpw = rpw // CHUNK
    wid = _wid()
    wbase = wid * rpw

    pltpu.sync_copy(mc.at[pl.ds(wbase, rpw)], ic)
    pltpu.sync_copy(mb.at[pl.ds(wbase, rpw)], ib)
    pltpu.sync_copy(ma.at[pl.ds(wbase, rpw)], ia)

    def start(j, rc, rb, ra, sem):
        off = j * CHUNK
        pltpu.async_copy(ct.at[ic.at[pl.ds(off, CHUNK)]], rc, sem)
        pltpu.async_copy(bt.at[ib.at[pl.ds(off, CHUNK)]], rb, sem)
        pltpu.async_copy(at.at[ia.at[pl.ds(off, CHUNK)]], ra, sem)

    def drain(j, rc, rb, ra, sem):
        base = wbase + j * CHUNK
        pltpu.make_async_copy(ct.at[ic.at[pl.ds(0, CHUNK)]], rc, sem).wait()
        pltpu.make_async_copy(bt.at[ib.at[pl.ds(0, CHUNK)]], rb, sem).wait()
        pltpu.make_async_copy(at.at[ia.at[pl.ds(0, CHUNK)]], ra, sem).wait()
        pltpu.sync_copy(rc, xc.at[pl.ds(base, CHUNK)])
        pltpu.sync_copy(rb, xb.at[pl.ds(base, CHUNK)])
        pltpu.sync_copy(ra, xa.at[pl.ds(base, CHUNK)])

    start(0, rc0, rb0, ra0, s0)

    def body(j, carry):
        @pl.when(j % 2 == 0)
        def _():
            @pl.when(j + 1 < cpw)
            def _():
                start(j + 1, rc1, rb1, ra1, s1)
            drain(j, rc0, rb0, ra0, s0)

        @pl.when(j % 2 == 1)
        def _():
            @pl.when(j + 1 < cpw)
            def _():
                start(j + 1, rc0, rb0, ra0, s0)
            drain(j, rc1, rb1, ra1, s1)

        return carry

    lax.fori_loop(0, cpw, body, 0)


def _mm_body(item_ref, xc_ref, xb_ref, xa_ref, w_ref, b_ref, h_ref):
    bf = jnp.bfloat16
    acc = jnp.dot(item_ref[...].astype(bf), w_ref[0:128, :].astype(bf),
                  preferred_element_type=jnp.float32)
    acc = acc + jnp.dot(xc_ref[:, 0:64].astype(bf), w_ref[128:192, :].astype(bf),
                        preferred_element_type=jnp.float32)
    acc = acc + jnp.dot(xb_ref[:, 0:64].astype(bf), w_ref[192:256, :].astype(bf),
                        preferred_element_type=jnp.float32)
    acc = acc + jnp.dot(xa_ref[:, 0:64].astype(bf), w_ref[256:320, :].astype(bf),
                        preferred_element_type=jnp.float32)
    h = jnp.tanh(acc + b_ref[...])
    row = lax.broadcasted_iota(jnp.int32, h.shape, 0)
    h_ref[...] = jnp.where((pl.program_id(0) == 0) & (row == 0), 0.0, h)


def _make_mm(bm):
    grid = NUM_IDS // bm
    return pl.pallas_call(
        _mm_body,
        grid=(grid,),
        in_specs=[
            pl.BlockSpec((bm, 128), lambda i: (i, 0)),
            pl.BlockSpec((bm, 128), lambda i: (i, 0)),
            pl.BlockSpec((bm, 128), lambda i: (i, 0)),
            pl.BlockSpec((bm, 128), lambda i: (i, 0)),
            pl.BlockSpec((320, FINAL), lambda i: (0, 0)),
            pl.BlockSpec((1, FINAL), lambda i: (0, 0)),
        ],
        out_specs=pl.BlockSpec((bm, FINAL), lambda i: (i, 0)),
        out_shape=jax.ShapeDtypeStruct((NUM_IDS, FINAL), jnp.float32),
    )


@functools.cache
def _make_emb_gather():
    return functools.partial(
        pl.kernel,
        mesh=plsc.VectorSubcoreMesh(**_MESH),
        out_type=jax.ShapeDtypeStruct((N_FLAT, FINAL), jnp.float32),
        scratch_types=[
            pltpu.VMEM((N_FLAT // NW,), jnp.int32),
            pltpu.VMEM((CHUNK, FINAL), jnp.float32),
            pltpu.VMEM((CHUNK, FINAL), jnp.float32),
            pltpu.SemaphoreType.DMA,
            pltpu.SemaphoreType.DMA,
        ],
    )(_emb_gather_body)


def _emb_gather_body(h, flat, out, idx, r0, r1, s0, s1):
    rpw = N_FLAT // NW
    cpw = rpw // CHUNK
    wid = _wid()
    wbase = wid * rpw

    pltpu.sync_copy(flat.at[pl.ds(wbase, rpw)], idx)

    def start(j, rows, sem):
        pltpu.async_copy(h.at[idx.at[pl.ds(j * CHUNK, CHUNK)]], rows, sem)

    def drain(j, rows, sem):
        pltpu.make_async_copy(h.at[idx.at[pl.ds(0, CHUNK)]], rows, sem).wait()
        pltpu.sync_copy(rows, out.at[pl.ds(wbase + j * CHUNK, CHUNK)])

    start(0, r0, s0)

    def body(j, carry):
        @pl.when(j % 2 == 0)
        def _():
            @pl.when(j + 1 < cpw)
            def _():
                start(j + 1, r1, s1)
            drain(j, r0, s0)

        @pl.when(j % 2 == 1)
        def _():
            @pl.when(j + 1 < cpw)
            def _():
                start(j + 1, r0, s0)
            drain(j, r1, s1)

        return carry

    lax.fori_loop(0, cpw, body, 0)


def kernel(sample, item_id_table, category_table, brand_table, author_table,
           map_category, map_brand, map_author, W, b):
    pad = PAD_IDS - NUM_IDS
    mc = jnp.pad(map_category, (0, pad))
    mb = jnp.pad(map_brand, (0, pad))
    ma = jnp.pad(map_author, (0, pad))
    ct = jnp.pad(category_table, ((0, 0), (0, 64)))
    bt = jnp.pad(brand_table, ((0, 0), (0, 64)))
    at = jnp.pad(author_table, ((0, 0), (0, 64)))
    xc, xb, xa = _make_attr_gather()(mc, mb, ma, ct, bt, at)
    h = _make_mm(800)(item_id_table, xc, xb, xa, W, b.reshape(1, FINAL))
    flat = sample.reshape(-1).astype(jnp.int32)
    out = _make_emb_gather()(h, flat)
    return out.reshape(sample.shape + (FINAL,))
